# baseline (speedup 1.0000x reference)
import functools

import jax
import jax.numpy as jnp
from jax import lax
from jax.experimental import pallas as pl
from jax.experimental.pallas import tpu as pltpu
from jax.experimental.pallas import tpu_sc as plsc

BATCH = 16384
DIM = 32
NROWS = 1000000
HALF = 524288
NC = 2
NS = 16
NW = NC * NS
BPW = BATCH // NW
CHUNK = 128
NCHUNK = BPW // CHUNK

BLK = 32768
NBLK = HALF // BLK
HB = HALF // BLK
LAST = (NROWS + BLK - 1) // BLK - 1


def _matvec_body(tma_ref, tmb_ref, tua_ref, tub_ref, wm_ref, wu_ref,
                 sma_ref, smb_ref, sua_ref, sub_ref):
  sma_ref[...] = jnp.dot(wm_ref[...], tma_ref[...],
                         preferred_element_type=jnp.float32)[0]
  smb_ref[...] = jnp.dot(wm_ref[...], tmb_ref[...],
                         preferred_element_type=jnp.float32)[0]
  sua_ref[...] = jnp.dot(wu_ref[...], tua_ref[...],
                         preferred_element_type=jnp.float32)[0]
  sub_ref[...] = jnp.dot(wu_ref[...], tub_ref[...],
                         preferred_element_type=jnp.float32)[0]


def _hi_map(i):
  return (0, jnp.minimum(i + HB, LAST))


def _matvec(tm, tu, wm, wu):
  return pl.pallas_call(
      _matvec_body,
      grid=(NBLK,),
      in_specs=[
          pl.BlockSpec((DIM, BLK), lambda i: (0, i)),
          pl.BlockSpec((DIM, BLK), _hi_map),
          pl.BlockSpec((DIM, BLK), lambda i: (0, i)),
          pl.BlockSpec((DIM, BLK), _hi_map),
          pl.BlockSpec((8, DIM), lambda i: (0, 0)),
          pl.BlockSpec((8, DIM), lambda i: (0, 0)),
      ],
      out_specs=[
          pl.BlockSpec((BLK,), lambda i: (i,)),
          pl.BlockSpec((BLK,), lambda i: (i,)),
          pl.BlockSpec((BLK,), lambda i: (i,)),
          pl.BlockSpec((BLK,), lambda i: (i,)),
      ],
      out_shape=[
          jax.ShapeDtypeStruct((HALF,), jnp.float32),
          jax.ShapeDtypeStruct((HALF,), jnp.float32),
          jax.ShapeDtypeStruct((HALF,), jnp.float32),
          jax.ShapeDtypeStruct((HALF,), jnp.float32),
      ],
  )(tm, tm, tu, tu, wm, wu)


def _sc_body(midx_hbm, uidx_hbm, sma_hbm, smb_hbm, sua_hbm, sub_hbm, bb_hbm,
             out_hbm, midx_v, uidx_v, midxl_v, uidxl_v,
             gma_v, gmb_v, gua_v, gub_v, bb_v, out_v, sem):
  wid = lax.axis_index("s") * NC + lax.axis_index("c")
  base = pl.multiple_of(wid * BPW, BPW)

  pltpu.sync_copy(midx_hbm.at[pl.ds(base, BPW)], midx_v)
  pltpu.sync_copy(uidx_hbm.at[pl.ds(base, BPW)], uidx_v)
  pltpu.sync_copy(bb_hbm, bb_v)

  for k in range(BPW // 16):
    sl = pl.ds(k * 16, 16)
    midxl_v[sl] = jnp.bitwise_and(midx_v[sl], HALF - 1)
    uidxl_v[sl] = jnp.bitwise_and(uidx_v[sl], HALF - 1)

  handles = []
  for j in range(NCHUNK):
    sl = pl.ds(j * CHUNK, CHUNK)
    handles.append(pltpu.async_copy(sma_hbm.at[midxl_v.at[sl]], gma_v.at[sl], sem))
    handles.append(pltpu.async_copy(smb_hbm.at[midxl_v.at[sl]], gmb_v.at[sl], sem))
    handles.append(pltpu.async_copy(sua_hbm.at[uidxl_v.at[sl]], gua_v.at[sl], sem))
    handles.append(pltpu.async_copy(sub_hbm.at[uidxl_v.at[sl]], gub_v.at[sl], sem))
  for h in handles:
    h.wait()

  bvec = bb_v[...]
  for k in range(BPW // 16):
    sl = pl.ds(k * 16, 16)
    vm = jnp.where(midx_v[sl] < HALF, gma_v[sl], gmb_v[sl])
    vu = jnp.where(uidx_v[sl] < HALF, gua_v[sl], gub_v[sl])
    out_v[sl] = vm + vu + bvec

  pltpu.sync_copy(out_v, out_hbm.at[pl.ds(base, BPW)])


def _sc_gather_add(midx, uidx, sma, smb, sua, sub, bb):
  mesh = plsc.VectorSubcoreMesh(core_axis_name="c", subcore_axis_name="s")
  return pl.kernel(
      _sc_body,
      out_type=jax.ShapeDtypeStruct((BATCH,), jnp.float32),
      mesh=mesh,
      scratch_types=[
          pltpu.VMEM((BPW,), jnp.int32),
          pltpu.VMEM((BPW,), jnp.int32),
          pltpu.VMEM((BPW,), jnp.int32),
          pltpu.VMEM((BPW,), jnp.int32),
          pltpu.VMEM((BPW,), jnp.float32),
          pltpu.VMEM((BPW,), jnp.float32),
          pltpu.VMEM((BPW,), jnp.float32),
          pltpu.VMEM((BPW,), jnp.float32),
          pltpu.VMEM((16,), jnp.float32),
          pltpu.VMEM((BPW,), jnp.float32),
          pltpu.SemaphoreType.DMA,
      ],
  )(midx, uidx, sma, smb, sua, sub, bb)


def kernel(user_index, movie_index, movie_table, user_table, W, b):
  tm = movie_table.T
  tu = user_table.T
  wm = jnp.zeros((8, DIM), jnp.float32).at[0].set(W[:DIM, 0])
  wu = jnp.zeros((8, DIM), jnp.float32).at[0].set(W[DIM:, 0])
  bb = jnp.broadcast_to(b.reshape(1), (16,)).astype(jnp.float32)
  sma, smb, sua, sub = _matvec(tm, tu, wm, wu)
  return _sc_gather_add(movie_index.astype(jnp.int32),
                        user_index.astype(jnp.int32), sma, smb, sua, sub, bb)
